# baseline (speedup 1.0000x reference)
import functools

import jax
import jax.numpy as jnp
from jax import lax
from jax.experimental import pallas as pl
from jax.experimental.pallas import tpu as pltpu
from jax.experimental.pallas import tpu_sc as plsc

_N_DRUGS = 2000
_N_CELLS = 1500
_B = 16384

_NC = 2
_NS = 16
_NW = _NC * _NS
_CHUNK = _B // _NW
_L = 16
_GSUB = 128
_NG = _CHUNK // _GSUB


def _make_sc_kernel():
    mesh = plsc.VectorSubcoreMesh(core_axis_name="c", subcore_axis_name="s")

    @functools.partial(
        pl.kernel,
        mesh=mesh,
        out_type=jax.ShapeDtypeStruct((_B,), jnp.float32),
        scratch_types=[
            pltpu.VMEM((_CHUNK,), jnp.float32),
            pltpu.VMEM((_CHUNK,), jnp.int32),
            pltpu.VMEM((_CHUNK,), jnp.int32),
            pltpu.VMEM((_CHUNK,), jnp.float32),
            pltpu.VMEM((_CHUNK,), jnp.float32),
            pltpu.VMEM((_CHUNK,), jnp.float32),
            pltpu.SemaphoreType.DMA,
        ],
    )
    def sc_kernel(x_hbm, did_hbm, cid_hbm, b_hbm, e_hbm, d_hbm, out_hbm,
                  xv, idxv, cidv, bv, ev, dv, sem):
        wid = lax.axis_index("s") * _NC + lax.axis_index("c")
        base = wid * _CHUNK

        pltpu.sync_copy(did_hbm.at[pl.ds(base, _CHUNK)], idxv)
        pltpu.sync_copy(cid_hbm.at[pl.ds(base, _CHUNK)], cidv)
        pltpu.sync_copy(x_hbm.at[pl.ds(base, _CHUNK)], xv)

        for i in range(_CHUNK // _L):
            sl = pl.ds(i * _L, _L)
            idxv[sl] = idxv[sl] * _N_CELLS + cidv[sl]

        copies = []
        for j in range(_NG):
            gs = pl.ds(j * _GSUB, _GSUB)
            idx_ref = idxv.at[gs]
            copies.append(pltpu.async_copy(b_hbm.at[idx_ref], bv.at[gs], sem))
            copies.append(pltpu.async_copy(e_hbm.at[idx_ref], ev.at[gs], sem))
            copies.append(pltpu.async_copy(d_hbm.at[idx_ref], dv.at[gs], sem))
        for c in copies:
            c.wait()

        for i in range(_CHUNK // _L):
            sl = pl.ds(i * _L, _L)
            t = bv[sl] * (xv[sl] + ev[sl])
            xv[sl] = dv[sl] / (1.0 + jnp.exp(-t))

        pltpu.sync_copy(xv, out_hbm.at[pl.ds(base, _CHUNK)])

    return sc_kernel


_sc_kernel = _make_sc_kernel()


@jax.jit
def kernel(x, drug_id, cell_id, b, e, d):
    bf = b.reshape(-1)
    ef = e.reshape(-1)
    df = d.reshape(-1)
    return _sc_kernel(x, drug_id.astype(jnp.int32), cell_id.astype(jnp.int32),
                      bf, ef, df)
